# baseline (speedup 1.0000x reference)
import jax
import jax.numpy as jnp
from jax import lax
from jax.experimental import pallas as pl
from jax.experimental.pallas import tpu as pltpu
from jax.experimental.pallas import tpu_sc as plsc

VOCAB = 1000000
EMBED_DIM = 64
BATCH = 16384
HIST = 20

_NC = 2
_NS = 16
_NW = _NC * _NS

_PER_W = BATCH // _NW
_CB = 4
_NOFF = _CB * HIST
_NVEC = _NOFF // 16
_NCHUNK = _PER_W // _CB
_NBUF = 8
_LAG = 4


def _body(idx_hbm, table_hbm, out_hbm, idx_v, idx_f, bufs, gsems, osems):
    wid = lax.axis_index("s") * _NC + lax.axis_index("c")
    base = wid * _PER_W

    pltpu.sync_copy(idx_hbm.at[pl.ds(base, _PER_W)], idx_v)

    lane = lax.iota(jnp.int32, 16)
    rowp = [(16 * k + lane) // HIST for k in range(_NVEC)]
    colp = [(16 * k + lane) % HIST for k in range(_NVEC)]

    def relayout(t, b):
        for k in range(_NVEC):
            vec = plsc.load_gather(idx_v, [t * _CB + rowp[k], colp[k]])
            idx_f[b, pl.ds(16 * k, 16)] = vec

    def gather(t, b):
        return pltpu.make_async_copy(
            table_hbm.at[idx_f.at[b]], bufs.at[b], gsems.at[b])

    def put_one(t, i, b):
        return pltpu.make_async_copy(
            bufs.at[b].at[pl.ds(i * HIST, HIST)],
            out_hbm.at[base + t * _CB + i], osems.at[b])

    def put_start(t, b):
        for i in range(_CB):
            put_one(t, i, b).start()

    def put_wait(t, b):
        for i in range(_CB):
            put_one(t, i, b).wait()

    for t in range(_NBUF):
        relayout(t, t % _NBUF)
        gather(t, t % _NBUF).start()
        s = t - _LAG
        if s >= 0:
            gather(s, s % _NBUF).wait()
            put_start(s, s % _NBUF)

    def step(t, carry):
        b = t % _NBUF
        put_wait(t - _NBUF, b)
        relayout(t, b)
        gather(t, b).start()
        s = t - _LAG
        bs = s % _NBUF
        gather(s, bs).wait()
        put_start(s, bs)
        return carry

    lax.fori_loop(_NBUF, _NCHUNK, step, 0)

    for s in range(_NCHUNK - _LAG, _NCHUNK):
        gather(s, s % _NBUF).wait()
        put_start(s, s % _NBUF)
    for s in range(_NCHUNK - _NBUF, _NCHUNK):
        put_wait(s, s % _NBUF)


@jax.jit
def _lookup(idx, table):
    mesh = plsc.VectorSubcoreMesh(core_axis_name="c", subcore_axis_name="s")
    k = pl.kernel(
        _body,
        out_type=jax.ShapeDtypeStruct((BATCH, HIST, EMBED_DIM), jnp.float32),
        mesh=mesh,
        scratch_types=dict(
            idx_v=pltpu.VMEM((_PER_W, HIST), jnp.int32),
            idx_f=pltpu.VMEM((_NBUF, _NOFF), jnp.int32),
            bufs=pltpu.VMEM((_NBUF, _NOFF, EMBED_DIM), jnp.float32),
            gsems=pltpu.SemaphoreType.DMA((_NBUF,)),
            osems=pltpu.SemaphoreType.DMA((_NBUF,)),
        ),
        compiler_params=pltpu.CompilerParams(
            use_tc_tiling_on_sc=False, needs_layout_passes=False),
    )
    return k(idx, table)


def kernel(id, W_center, W_context):
    return _lookup(id.astype(jnp.int32), W_center)


# speedup vs baseline: 1.1873x; 1.1873x over previous
import jax
import jax.numpy as jnp
from jax import lax
from jax.experimental import pallas as pl
from jax.experimental.pallas import tpu as pltpu
from jax.experimental.pallas import tpu_sc as plsc

VOCAB = 1000000
EMBED_DIM = 64
BATCH = 16384
HIST = 20

_NC = 2
_NS = 16
_NW = _NC * _NS

_PER_W = BATCH // _NW
_CB = 4
_NOFF = _CB * HIST
_NVEC = _NOFF // 16
_NCHUNK = _PER_W // _CB
_NBUF = 8
_LAG = 4


def _body(idx_hbm, table_hbm, out_hbm, idx_v, idx_f, bufs, gsems, osems):
    wid = lax.axis_index("s") * _NC + lax.axis_index("c")
    base = wid * _PER_W

    pltpu.sync_copy(idx_hbm.at[pl.ds(base, _PER_W)], idx_v)

    lane = lax.iota(jnp.int32, 16)
    rowp = [(16 * k + lane) // HIST for k in range(_NVEC)]
    colp = [(16 * k + lane) % HIST for k in range(_NVEC)]

    def relayout(t, b):
        for k in range(_NVEC):
            vec = plsc.load_gather(idx_v, [t * _CB + rowp[k], colp[k]])
            idx_f[b, pl.ds(16 * k, 16)] = vec

    def gather(t, b):
        return pltpu.make_async_copy(
            table_hbm.at[idx_f.at[b]], bufs.at[b], gsems.at[b])

    def put_one(t, i, b):
        return pltpu.make_async_copy(
            bufs.at[b].at[pl.ds(i * HIST, HIST)],
            out_hbm.at[base + t * _CB + i].at[pl.ds(0, HIST), pl.ds(0, EMBED_DIM)],
            osems.at[b])

    def put_start(t, b):
        for i in range(_CB):
            put_one(t, i, b).start()

    def put_wait(t, b):
        for i in range(_CB):
            put_one(t, i, b).wait()

    for t in range(_NBUF):
        relayout(t, t % _NBUF)
        gather(t, t % _NBUF).start()
        s = t - _LAG
        if s >= 0:
            gather(s, s % _NBUF).wait()
            put_start(s, s % _NBUF)

    def step(t, carry):
        b = t % _NBUF
        put_wait(t - _NBUF, b)
        relayout(t, b)
        gather(t, b).start()
        s = t - _LAG
        bs = s % _NBUF
        gather(s, bs).wait()
        put_start(s, bs)
        return carry

    lax.fori_loop(_NBUF, _NCHUNK, step, 0)

    for s in range(_NCHUNK - _LAG, _NCHUNK):
        gather(s, s % _NBUF).wait()
        put_start(s, s % _NBUF)
    for s in range(_NCHUNK - _NBUF, _NCHUNK):
        put_wait(s, s % _NBUF)


@jax.jit
def _lookup(idx, table):
    mesh = plsc.VectorSubcoreMesh(core_axis_name="c", subcore_axis_name="s")
    k = pl.kernel(
        _body,
        out_type=jax.ShapeDtypeStruct((BATCH, 24, 128), jnp.float32),
        mesh=mesh,
        scratch_types=dict(
            idx_v=pltpu.VMEM((_PER_W, HIST), jnp.int32),
            idx_f=pltpu.VMEM((_NBUF, _NOFF), jnp.int32),
            bufs=pltpu.VMEM((_NBUF, _NOFF, EMBED_DIM), jnp.float32),
            gsems=pltpu.SemaphoreType.DMA((_NBUF,)),
            osems=pltpu.SemaphoreType.DMA((_NBUF,)),
        ),
        compiler_params=pltpu.CompilerParams(
            use_tc_tiling_on_sc=False, needs_layout_passes=False),
    )
    return k(idx, table)


def kernel(id, W_center, W_context):
    return _lookup(id.astype(jnp.int32), W_center)[:, :HIST, :EMBED_DIM]


# speedup vs baseline: 1.2467x; 1.0500x over previous
import jax
import jax.numpy as jnp
from jax import lax
from jax.experimental import pallas as pl
from jax.experimental.pallas import tpu as pltpu
from jax.experimental.pallas import tpu_sc as plsc

VOCAB = 1000000
EMBED_DIM = 64
BATCH = 16384
HIST = 20

_NC = 2
_NS = 16
_NW = _NC * _NS

_PER_W = BATCH // _NW
_CB = 4
_NOFF = _CB * HIST
_NVEC = _NOFF // 16
_NCHUNK = _PER_W // _CB
_NBUF = 8
_LAG = 4


def _body(idx_hbm, table_hbm, out_hbm, idx_v, idx_f, bufs, gsems, osems):
    wid = lax.axis_index("s") * _NC + lax.axis_index("c")
    base = wid * _PER_W

    pltpu.sync_copy(idx_hbm.at[pl.ds(base, _PER_W)], idx_v)

    lane = lax.iota(jnp.int32, 16)
    rowp = [(16 * k + lane) // HIST for k in range(_NVEC)]
    colp = [(16 * k + lane) % HIST for k in range(_NVEC)]

    def relayout(t, b):
        for k in range(_NVEC):
            vec = plsc.load_gather(idx_v, [t * _CB + rowp[k], colp[k]])
            idx_f[b, pl.ds(16 * k, 16)] = vec

    def gather(t, b):
        return pltpu.make_async_copy(
            table_hbm.at[idx_f.at[b]], bufs.at[b], gsems.at[b])

    def put_one(t, i, b):
        return pltpu.make_async_copy(
            bufs.at[b].at[pl.ds(i * HIST, HIST), pl.ds(0, EMBED_DIM)],
            out_hbm.at[base + t * _CB + i].at[pl.ds(0, HIST), pl.ds(0, EMBED_DIM)],
            osems.at[b])

    def put_start(t, b):
        for i in range(_CB):
            put_one(t, i, b).start()

    def put_wait(t, b):
        for i in range(_CB):
            put_one(t, i, b).wait()

    for t in range(_NBUF):
        relayout(t, t % _NBUF)
        gather(t, t % _NBUF).start()
        s = t - _LAG
        if s >= 0:
            gather(s, s % _NBUF).wait()
            put_start(s, s % _NBUF)

    def step(t, carry):
        b = t % _NBUF
        put_wait(t - _NBUF, b)
        relayout(t, b)
        gather(t, b).start()
        s = t - _LAG
        bs = s % _NBUF
        gather(s, bs).wait()
        put_start(s, bs)
        return carry

    lax.fori_loop(_NBUF, _NCHUNK, step, 0)

    for s in range(_NCHUNK - _LAG, _NCHUNK):
        gather(s, s % _NBUF).wait()
        put_start(s, s % _NBUF)
    for s in range(_NCHUNK - _NBUF, _NCHUNK):
        put_wait(s, s % _NBUF)


@jax.jit
def _lookup(idx, table):
    mesh = plsc.VectorSubcoreMesh(core_axis_name="c", subcore_axis_name="s")
    k = pl.kernel(
        _body,
        out_type=jax.ShapeDtypeStruct((BATCH, 24, 128), jnp.float32),
        mesh=mesh,
        scratch_types=dict(
            idx_v=pltpu.VMEM((_PER_W, HIST), jnp.int32),
            idx_f=pltpu.VMEM((_NBUF, _NOFF), jnp.int32),
            bufs=pltpu.VMEM((_NBUF, _NOFF, 128), jnp.float32),
            gsems=pltpu.SemaphoreType.DMA((_NBUF,)),
            osems=pltpu.SemaphoreType.DMA((_NBUF,)),
        ),
        compiler_params=pltpu.CompilerParams(
            use_tc_tiling_on_sc=False, needs_layout_passes=False),
    )
    return k(idx, table)


def kernel(id, W_center, W_context):
    table = jnp.pad(W_center, ((0, 0), (0, 128 - EMBED_DIM)))
    return _lookup(id.astype(jnp.int32), table)[:, :HIST, :EMBED_DIM]


# speedup vs baseline: 2.0576x; 1.6504x over previous
import jax
import jax.numpy as jnp
from jax import lax
from jax.experimental import pallas as pl
from jax.experimental.pallas import tpu as pltpu
from jax.experimental.pallas import tpu_sc as plsc

VOCAB = 1000000
EMBED_DIM = 64
BATCH = 16384
HIST = 20

_NC = 2
_NS = 16
_NW = _NC * _NS

_PER_W = BATCH // _NW
_CB = 4
_NOFF = _CB * HIST
_NVEC = _NOFF // 16
_NCHUNK = _PER_W // _CB
_NBUF = 8
_LAG = 4


def _body(idx_hbm, table_hbm, out_hbm, idx_v, idx_f, bufs, gsems, osems):
    wid = lax.axis_index("s") * _NC + lax.axis_index("c")
    base = wid * _PER_W

    pltpu.sync_copy(idx_hbm.at[pl.ds(base, _PER_W)], idx_v)

    lane = lax.iota(jnp.int32, 16)
    rowp = [(16 * k + lane) // HIST for k in range(_NVEC)]
    colp = [(16 * k + lane) % HIST for k in range(_NVEC)]

    def relayout(t, b):
        for k in range(_NVEC):
            vec = plsc.load_gather(idx_v, [t * _CB + rowp[k], colp[k]])
            idx_f[b, pl.ds(16 * k, 16)] = vec

    def gather(t, b):
        return pltpu.make_async_copy(
            table_hbm.at[idx_f.at[b]], bufs.at[b], gsems.at[b])

    def put_one(t, i, b):
        return pltpu.make_async_copy(
            bufs.at[b].at[pl.ds(i * HIST, HIST), pl.ds(0, EMBED_DIM)],
            out_hbm.at[base + t * _CB + i].at[pl.ds(0, HIST), pl.ds(0, EMBED_DIM)],
            osems.at[b])

    def put_start(t, b):
        for i in range(_CB):
            put_one(t, i, b).start()

    def put_wait(t, b):
        for i in range(_CB):
            put_one(t, i, b).wait()

    for t in range(_NBUF):
        relayout(t, t % _NBUF)
        gather(t, t % _NBUF).start()
        s = t - _LAG
        if s >= 0:
            gather(s, s % _NBUF).wait()
            put_start(s, s % _NBUF)

    def step(t, carry):
        b = t % _NBUF
        put_wait(t - _NBUF, b)
        relayout(t, b)
        gather(t, b).start()
        s = t - _LAG
        bs = s % _NBUF
        gather(s, bs).wait()
        put_start(s, bs)
        return carry

    lax.fori_loop(_NBUF, _NCHUNK, step, 0)

    for s in range(_NCHUNK - _LAG, _NCHUNK):
        gather(s, s % _NBUF).wait()
        put_start(s, s % _NBUF)
    for s in range(_NCHUNK - _NBUF, _NCHUNK):
        put_wait(s, s % _NBUF)


@jax.jit
def _lookup(idx, table):
    mesh = plsc.VectorSubcoreMesh(core_axis_name="c", subcore_axis_name="s")
    k = pl.kernel(
        _body,
        out_type=jax.ShapeDtypeStruct((BATCH, 24, 128), jnp.float32),
        mesh=mesh,
        scratch_types=dict(
            idx_v=pltpu.VMEM((_PER_W, HIST), jnp.int32),
            idx_f=pltpu.VMEM((_NBUF, _NOFF), jnp.int32),
            bufs=pltpu.VMEM((_NBUF, _NOFF, 128), jnp.float32),
            gsems=pltpu.SemaphoreType.DMA((_NBUF,)),
            osems=pltpu.SemaphoreType.DMA((_NBUF,)),
        ),
        compiler_params=pltpu.CompilerParams(
            use_tc_tiling_on_sc=False, needs_layout_passes=False),
    )
    return k(idx, table)


def kernel(id, W_center, W_context):
    proj = jnp.eye(EMBED_DIM, 128, dtype=jnp.float32)
    table = W_center @ proj
    return _lookup(id.astype(jnp.int32), table)[:, :HIST, :EMBED_DIM]
